# baseline (speedup 1.0000x reference)
import functools

import numpy as np

import jax
import jax.numpy as jnp
from jax import lax
from jax.experimental import pallas as pl
from jax.experimental.pallas import tpu as pltpu

N = 10000
E = 160000
F_IN = 300
HEADS = 2
F_HEAD = 300
F_OUT = 600
CLASSES = 500

NP = 10240
FP = 640
CP = 512
SCALE = 1.0 / np.sqrt(float(F_OUT))

BI = 512
BJ = 512
NI = NP // BI
NJ = NP // BJ


def _q_body(h_ref, w_ref, q_ref):
    q_ref[...] = jnp.dot(h_ref[...], w_ref[...], preferred_element_type=jnp.float32)


def _stats_body(q_ref, h_ref, s_ref, m_ref, z_ref, m_acc, z_acc, *, ni, bi):
    i = pl.program_id(1)
    s = jax.lax.dot_general(q_ref[...], h_ref[...], (((1,), (1,)), ((), ())),
                            preferred_element_type=jnp.float32) * SCALE
    row = i * bi + lax.broadcasted_iota(jnp.int32, s.shape, 0)
    s = jnp.where(row < N, s, -1e30)
    s_ref[...] = s
    bmax = jnp.max(s, axis=0, keepdims=True)

    @pl.when(i == 0)
    def _init():
        m_acc[...] = bmax
        z_acc[...] = jnp.sum(jnp.exp(s - bmax), axis=0, keepdims=True)

    @pl.when(i > 0)
    def _update():
        m_new = jnp.maximum(m_acc[...], bmax)
        z_acc[...] = (z_acc[...] * jnp.exp(m_acc[...] - m_new)
                      + jnp.sum(jnp.exp(s - m_new), axis=0, keepdims=True))
        m_acc[...] = m_new

    @pl.when(i == ni - 1)
    def _fin():
        m_ref[...] = m_acc[...]
        z_ref[...] = z_acc[...]


def _attn_body(s_ref, m_ref, v_ref, wl_ref, bl_ref, o_ref, acc, *, nj):
    j = pl.program_id(1)
    p = jnp.exp(s_ref[...] - m_ref[...])
    pv = jnp.dot(p, v_ref[...], preferred_element_type=jnp.float32)

    @pl.when(j == 0)
    def _init():
        acc[...] = pv

    @pl.when(j > 0)
    def _update():
        acc[...] = acc[...] + pv

    @pl.when(j == nj - 1)
    def _fin():
        logits = jnp.dot(acc[...], wl_ref[...], preferred_element_type=jnp.float32)
        logits = logits + bl_ref[...]
        o_ref[...] = jax.nn.softmax(logits, axis=-1)


def _attention(hpad, W_att_pad, W_lin_pad, b_lin_pad):
    bn = 1024
    q = pl.pallas_call(
        _q_body,
        grid=(NP // bn,),
        in_specs=[
            pl.BlockSpec((bn, FP), lambda i: (i, 0)),
            pl.BlockSpec((FP, FP), lambda i: (0, 0)),
        ],
        out_specs=pl.BlockSpec((bn, FP), lambda i: (i, 0)),
        out_shape=jax.ShapeDtypeStruct((NP, FP), jnp.float32),
    )(hpad, W_att_pad)

    s, m, z = pl.pallas_call(
        functools.partial(_stats_body, ni=NI, bi=BI),
        grid=(NJ, NI),
        in_specs=[
            pl.BlockSpec((BI, FP), lambda j, i: (i, 0)),
            pl.BlockSpec((BJ, FP), lambda j, i: (j, 0)),
        ],
        out_specs=[
            pl.BlockSpec((BI, BJ), lambda j, i: (i, j)),
            pl.BlockSpec((1, BJ), lambda j, i: (0, j)),
            pl.BlockSpec((1, BJ), lambda j, i: (0, j)),
        ],
        out_shape=[
            jax.ShapeDtypeStruct((NP, NP), jnp.float32),
            jax.ShapeDtypeStruct((1, NP), jnp.float32),
            jax.ShapeDtypeStruct((1, NP), jnp.float32),
        ],
        scratch_shapes=[
            pltpu.VMEM((1, BJ), jnp.float32),
            pltpu.VMEM((1, BJ), jnp.float32),
        ],
        compiler_params=pltpu.CompilerParams(
            dimension_semantics=("parallel", "arbitrary")),
    )(q, hpad)

    vprime = hpad / z.reshape(NP, 1)

    out_pad = pl.pallas_call(
        functools.partial(_attn_body, nj=NJ),
        grid=(NI, NJ),
        in_specs=[
            pl.BlockSpec((BI, BJ), lambda i, j: (i, j)),
            pl.BlockSpec((1, BJ), lambda i, j: (0, j)),
            pl.BlockSpec((BJ, FP), lambda i, j: (j, 0)),
            pl.BlockSpec((FP, CP), lambda i, j: (0, 0)),
            pl.BlockSpec((1, CP), lambda i, j: (0, 0)),
        ],
        out_specs=pl.BlockSpec((BI, CP), lambda i, j: (i, 0)),
        out_shape=jax.ShapeDtypeStruct((NP, CP), jnp.float32),
        scratch_shapes=[pltpu.VMEM((BI, FP), jnp.float32)],
        compiler_params=pltpu.CompilerParams(
            dimension_semantics=("parallel", "arbitrary")),
    )(s, m, vprime, W_lin_pad, b_lin_pad.reshape(1, CP))
    return out_pad


def kernel(features, edges, W_gat, a_src, a_dst, b_gat, W_att, W_lin, b_lin):
    x = features
    src = edges[0]
    dst = edges[1]

    xp = jnp.einsum('nf,hfo->nho', x, W_gat)
    alpha_src = jnp.sum(xp * a_src[None, :, :], axis=-1)
    alpha_dst = jnp.sum(xp * a_dst[None, :, :], axis=-1)
    e = jax.nn.leaky_relu(alpha_src[src] + alpha_dst[dst], negative_slope=0.2)
    ee = jnp.exp(e)
    denom = jax.ops.segment_sum(ee, dst, num_segments=N)
    denom = jnp.where(denom > 0, denom, 1.0)
    alpha = ee / denom[dst]
    msg = xp[src] * alpha[:, :, None]
    out = jax.ops.segment_sum(msg, dst, num_segments=N)
    h = out.reshape(N, F_OUT) + b_gat

    idx600 = jnp.concatenate([jnp.arange(300), 320 + jnp.arange(300)])
    hpad = jnp.zeros((NP, FP), jnp.float32).at[:N, idx600].set(h)
    W_att_pad = jnp.zeros((FP, FP), jnp.float32).at[idx600[:, None], idx600[None, :]].set(W_att)
    W_lin_pad = jnp.zeros((FP, CP), jnp.float32).at[idx600, :CLASSES].set(W_lin)
    b_lin_pad = jnp.full((CP,), -1e30, jnp.float32).at[:CLASSES].set(b_lin)

    out_pad = _attention(hpad, W_att_pad, W_lin_pad, b_lin_pad)
    return out_pad[:N, :CLASSES]


# speedup vs baseline: 1.4746x; 1.4746x over previous
import functools

import numpy as np

import jax
import jax.numpy as jnp
from jax import lax
from jax.experimental import pallas as pl
from jax.experimental.pallas import tpu as pltpu

N = 10000
E = 160000
F_IN = 300
HEADS = 2
F_HEAD = 300
F_OUT = 600
CLASSES = 500

NP = 10240
FP = 640
CP = 512
SCALE = 1.0 / np.sqrt(float(F_OUT))

BI = 512
BJ = 512
NI = NP // BI
NJ = NP // BJ


def _q_body(h_ref, w_ref, q_ref):
    q_ref[...] = jnp.dot(h_ref[...], w_ref[...], preferred_element_type=jnp.float32)


def _stats_body(q_ref, h_ref, s_ref, m_ref, z_ref, m_acc, z_acc, *, ni, bi):
    i = pl.program_id(1)
    s = jax.lax.dot_general(q_ref[...], h_ref[...], (((1,), (1,)), ((), ())),
                            preferred_element_type=jnp.float32) * SCALE
    row = i * bi + lax.broadcasted_iota(jnp.int32, s.shape, 0)
    s = jnp.where(row < N, s, -1e30)
    s_ref[...] = s
    bmax = jnp.max(s, axis=0, keepdims=True)

    @pl.when(i == 0)
    def _init():
        m_acc[...] = bmax
        z_acc[...] = jnp.sum(jnp.exp(s - bmax), axis=0, keepdims=True)

    @pl.when(i > 0)
    def _update():
        m_new = jnp.maximum(m_acc[...], bmax)
        z_acc[...] = (z_acc[...] * jnp.exp(m_acc[...] - m_new)
                      + jnp.sum(jnp.exp(s - m_new), axis=0, keepdims=True))
        m_acc[...] = m_new

    @pl.when(i == ni - 1)
    def _fin():
        m_ref[...] = m_acc[...]
        z_ref[...] = z_acc[...]


def _attn_body(s_ref, m_ref, v_ref, wl_ref, bl_ref, o_ref, acc, *, nj):
    j = pl.program_id(1)
    p = jnp.exp(s_ref[...] - m_ref[...])
    pv = jnp.dot(p, v_ref[...], preferred_element_type=jnp.float32)

    @pl.when(j == 0)
    def _init():
        acc[...] = pv

    @pl.when(j > 0)
    def _update():
        acc[...] = acc[...] + pv

    @pl.when(j == nj - 1)
    def _fin():
        logits = jnp.dot(acc[...], wl_ref[...], preferred_element_type=jnp.float32)
        logits = logits + bl_ref[...]
        o_ref[...] = jax.nn.softmax(logits, axis=-1)


def _attention(hpad, W_att_pad, W_lin_pad, b_lin_pad):
    bn = 1024
    q = pl.pallas_call(
        _q_body,
        grid=(NP // bn,),
        in_specs=[
            pl.BlockSpec((bn, FP), lambda i: (i, 0)),
            pl.BlockSpec((FP, FP), lambda i: (0, 0)),
        ],
        out_specs=pl.BlockSpec((bn, FP), lambda i: (i, 0)),
        out_shape=jax.ShapeDtypeStruct((NP, FP), jnp.float32),
    )(hpad, W_att_pad)

    s, m, z = pl.pallas_call(
        functools.partial(_stats_body, ni=NI, bi=BI),
        grid=(NJ, NI),
        in_specs=[
            pl.BlockSpec((BI, FP), lambda j, i: (i, 0)),
            pl.BlockSpec((BJ, FP), lambda j, i: (j, 0)),
        ],
        out_specs=[
            pl.BlockSpec((BI, BJ), lambda j, i: (i, j)),
            pl.BlockSpec((1, BJ), lambda j, i: (0, j)),
            pl.BlockSpec((1, BJ), lambda j, i: (0, j)),
        ],
        out_shape=[
            jax.ShapeDtypeStruct((NP, NP), jnp.float32),
            jax.ShapeDtypeStruct((1, NP), jnp.float32),
            jax.ShapeDtypeStruct((1, NP), jnp.float32),
        ],
        scratch_shapes=[
            pltpu.VMEM((1, BJ), jnp.float32),
            pltpu.VMEM((1, BJ), jnp.float32),
        ],
        compiler_params=pltpu.CompilerParams(
            dimension_semantics=("parallel", "arbitrary")),
    )(q, hpad)

    vprime = hpad / z.reshape(NP, 1)

    out_pad = pl.pallas_call(
        functools.partial(_attn_body, nj=NJ),
        grid=(NI, NJ),
        in_specs=[
            pl.BlockSpec((BI, BJ), lambda i, j: (i, j)),
            pl.BlockSpec((1, BJ), lambda i, j: (0, j)),
            pl.BlockSpec((BJ, FP), lambda i, j: (j, 0)),
            pl.BlockSpec((FP, CP), lambda i, j: (0, 0)),
            pl.BlockSpec((1, CP), lambda i, j: (0, 0)),
        ],
        out_specs=pl.BlockSpec((BI, CP), lambda i, j: (i, 0)),
        out_shape=jax.ShapeDtypeStruct((NP, CP), jnp.float32),
        scratch_shapes=[pltpu.VMEM((BI, FP), jnp.float32)],
        compiler_params=pltpu.CompilerParams(
            dimension_semantics=("parallel", "arbitrary")),
    )(s, m, vprime, W_lin_pad, b_lin_pad.reshape(1, CP))
    return out_pad


def kernel(features, edges, W_gat, a_src, a_dst, b_gat, W_att, W_lin, b_lin):
    x = features
    src = edges[0]
    dst = edges[1]

    xp = jnp.einsum('nf,hfo->nho', x, W_gat)
    alpha_src = jnp.sum(xp * a_src[None, :, :], axis=-1)
    alpha_dst = jnp.sum(xp * a_dst[None, :, :], axis=-1)
    e = jax.nn.leaky_relu(alpha_src[src] + alpha_dst[dst], negative_slope=0.2)
    ee = jnp.exp(e)
    denom = jax.ops.segment_sum(ee, dst, num_segments=N)
    denom = jnp.where(denom > 0, denom, 1.0)
    alpha = ee / denom[dst]
    msg = xp[src] * alpha[:, :, None]
    out = jax.ops.segment_sum(msg, dst, num_segments=N)
    h = out.reshape(N, F_OUT) + b_gat

    hpad = jnp.pad(h.reshape(N, 2, 300), ((0, NP - N), (0, 0), (0, 20))).reshape(NP, FP)
    W_att_pad = jnp.pad(W_att.reshape(2, 300, 2, 300),
                        ((0, 0), (0, 20), (0, 0), (0, 20))).reshape(FP, FP)
    W_lin_pad = jnp.pad(W_lin.reshape(2, 300, CLASSES),
                        ((0, 0), (0, 20), (0, CP - CLASSES))).reshape(FP, CP)
    b_lin_pad = jnp.concatenate([b_lin, jnp.full((CP - CLASSES,), -1e30, jnp.float32)])

    out_pad = _attention(hpad, W_att_pad, W_lin_pad, b_lin_pad)
    return out_pad[:N, :CLASSES]


# speedup vs baseline: 8.3416x; 5.6569x over previous
import functools

import numpy as np

import jax
import jax.numpy as jnp
from jax import lax
from jax.experimental import pallas as pl
from jax.experimental.pallas import tpu as pltpu
from jax.experimental.pallas import tpu_sc as plsc

N = 10000
E = 160000
F_OUT = 600
CLASSES = 500

NP = 10240
FP = 640
CP = 512
SCALE = 1.0 / np.sqrt(float(F_OUT))

EP = 163840
EPW = EP // 32
NG = EPW // 16
DTOT = 2 * NP
ATOT = 4 * NP
NPASS = 4
OWN = NP // 32
PR = OWN // NPASS
CBUF = EPW + 16

BI = 512
BJ = 512
NI = NP // BI
NJ = NP // BJ

_GD = lax.GatherDimensionNumbers(
    offset_dims=(), collapsed_slice_dims=(0,), start_index_map=(0,))


def _bcast_lane(v, lane):
    idx = jnp.broadcast_to(lane.astype(jnp.int32), (16,))
    return lax.gather(v, idx[:, None], _GD, slice_sizes=(1,),
                      mode=lax.GatherScatterMode.PROMISE_IN_BOUNDS)


def _bcast_lane_i32(v, lane):
    return _bcast_lane(v, lane)


def _exp_sc(x):
    t = jnp.clip(x * 1.4426950408889634, -125.0, 125.0)
    n = (t + 12582912.0) - 12582912.0
    f = t - n
    p = 0.0001540353039338161
    p = p * f + 0.0013333558146428443
    p = p * f + 0.009618129107628477
    p = p * f + 0.05550410866482158
    p = p * f + 0.2402265069591007
    p = p * f + 0.6931471805599453
    p = p * f + 1.0
    bits = (n.astype(jnp.int32) + 127) << 23
    return plsc.bitcast(bits, jnp.float32) * p


def _xp_body(x_ref, w_ref, a_ref, xp_ref, al_ref):
    xp = jnp.dot(x_ref[...], w_ref[...], preferred_element_type=jnp.float32)
    xp_ref[...] = xp
    al_ref[...] = jnp.dot(xp, a_ref[...], preferred_element_type=jnp.float32)


def _hq_body(p_ref, dn_ref, b_ref, w_ref, h_ref, q_ref, *, bn):
    i = pl.program_id(0)
    rec = 1.0 / dn_ref[...]
    rec640 = jnp.broadcast_to(rec[:, :, None], (bn, 2, FP // 2)).reshape(bn, FP)
    h = p_ref[...] * rec640 + b_ref[...]
    row = i * bn + lax.broadcasted_iota(jnp.int32, h.shape, 0)
    h = jnp.where(row < N, h, 0.0)
    h_ref[...] = h
    q_ref[...] = jnp.dot(h, w_ref[...], preferred_element_type=jnp.float32)


def _stats_body(q_ref, h_ref, s_ref, m_ref, z_ref, m_acc, z_acc, *, ni, bi):
    i = pl.program_id(1)
    s = lax.dot_general(q_ref[...], h_ref[...], (((1,), (1,)), ((), ())),
                        preferred_element_type=jnp.float32) * SCALE
    row = i * bi + lax.broadcasted_iota(jnp.int32, s.shape, 0)
    s = jnp.where(row < N, s, -1e30)
    s_ref[...] = s
    bmax = jnp.max(s, axis=0, keepdims=True)

    @pl.when(i == 0)
    def _init():
        m_acc[...] = bmax
        z_acc[...] = jnp.sum(jnp.exp(s - bmax), axis=0, keepdims=True)

    @pl.when(i > 0)
    def _update():
        m_new = jnp.maximum(m_acc[...], bmax)
        z_acc[...] = (z_acc[...] * jnp.exp(m_acc[...] - m_new)
                      + jnp.sum(jnp.exp(s - m_new), axis=0, keepdims=True))
        m_acc[...] = m_new

    @pl.when(i == ni - 1)
    def _fin():
        m_ref[...] = m_acc[...]
        z_ref[...] = z_acc[...]


def _attn_body(s_ref, m_ref, v_ref, wl_ref, bl_ref, o_ref, acc, *, nj):
    j = pl.program_id(1)
    p = jnp.exp(s_ref[...] - m_ref[...])
    pv = jnp.dot(p, v_ref[...], preferred_element_type=jnp.float32)

    @pl.when(j == 0)
    def _init():
        acc[...] = pv

    @pl.when(j > 0)
    def _update():
        acc[...] = acc[...] + pv

    @pl.when(j == nj - 1)
    def _fin():
        logits = jnp.dot(acc[...], wl_ref[...], preferred_element_type=jnp.float32)
        logits = logits + bl_ref[...]
        o_ref[...] = jax.nn.softmax(logits, axis=-1)


def _edge_ee_body(src_hbm, dst_hbm, atab_hbm, ed_hbm, dn_hbm,
                  srcv, dstv, ee0v, ee1v, atab, dnv, sem):
    c = lax.axis_index("c")
    s = lax.axis_index("s")
    wid = c * 16 + s
    base = wid * EPW
    pltpu.sync_copy(src_hbm.at[pl.ds(base, EPW)], srcv)
    pltpu.sync_copy(dst_hbm.at[pl.ds(base, EPW)], dstv)
    pltpu.sync_copy(atab_hbm, atab)
    z16 = jnp.zeros((16,), jnp.float32)

    def zbody(i, _):
        dnv[pl.ds(i * 16, 16)] = z16
        return 0
    lax.fori_loop(0, DTOT // 16, zbody, 0)

    lanes = jnp.arange(16, dtype=jnp.int32)

    def gbody(g, _):
        sv = srcv[pl.ds(g * 16, 16)]
        dv = dstv[pl.ds(g * 16, 16)]
        s4 = sv * 4
        d4 = dv * 4
        as0 = plsc.load_gather(atab, [s4])
        as1 = plsc.load_gather(atab, [s4 + 1])
        ad0 = plsc.load_gather(atab, [d4 + 2])
        ad1 = plsc.load_gather(atab, [d4 + 3])
        e0 = as0 + ad0
        e1 = as1 + ad1
        x0 = _exp_sc(jnp.where(e0 >= 0, e0, 0.2 * e0))
        x1 = _exp_sc(jnp.where(e1 >= 0, e1, 0.2 * e1))
        ee0v[pl.ds(g * 16, 16)] = plsc.bitcast(x0, jnp.int32)
        ee1v[pl.ds(g * 16, 16)] = plsc.bitcast(x1, jnp.int32)
        for l in range(16):
            m = lanes == l
            plsc.addupdate_scatter(dnv, [dv], x0, mask=m)
            plsc.addupdate_scatter(dnv, [dv + NP], x1, mask=m)
        return 0
    lax.fori_loop(0, NG, gbody, 0)

    eb = wid * 4 * EPW
    pltpu.sync_copy(srcv, ed_hbm.at[pl.ds(eb, EPW)])
    pltpu.sync_copy(dstv, ed_hbm.at[pl.ds(eb + EPW, EPW)])
    pltpu.sync_copy(ee0v, ed_hbm.at[pl.ds(eb + 2 * EPW, EPW)])
    pltpu.sync_copy(ee1v, ed_hbm.at[pl.ds(eb + 3 * EPW, EPW)])
    pltpu.sync_copy(dnv, dn_hbm.at[pl.ds(wid * DTOT, DTOT)])


def _denom_reduce_body(dn_hbm, out_hbm, buf, accv, sem):
    c = lax.axis_index("c")
    s = lax.axis_index("s")
    wid = c * 16 + s
    seg = DTOT // 32
    off = wid * seg
    for r in range(32):
        pltpu.sync_copy(dn_hbm.at[pl.ds(r * DTOT + off, seg)],
                        buf.at[pl.ds(r * seg, seg)])

    def cbody(cc, _):
        acc = buf[pl.ds(cc * 16, 16)]
        for r in range(1, 32):
            acc = acc + buf[pl.ds(r * seg + cc * 16, 16)]
        accv[pl.ds(cc * 16, 16)] = jnp.where(acc > 0, acc, 1.0)
        return 0
    lax.fori_loop(0, seg // 16, cbody, 0)
    pltpu.sync_copy(accv, out_hbm.at[pl.ds(off, seg)])


def _msg_body(ed_hbm, xp_hbm, part_hbm,
              sbuf, csrcf, cdstf, ca0, ca1, acc, rows, sem):
    c = lax.axis_index("c")
    s = lax.axis_index("s")
    wid = c * 16 + s
    node_lo = wid * OWN

    z16 = jnp.zeros((16,), jnp.float32)
    iz16 = jnp.zeros((16,), jnp.int32)
    lanes = jnp.arange(16, dtype=jnp.int32)

    for p in range(NPASS):
        lo = node_lo + p * PR

        def zcbody(i, _):
            acc[i // (FP // 16), pl.ds((i % (FP // 16)) * 16, 16)] = z16
            return 0
        lax.fori_loop(0, PR * (FP // 16), zcbody, 0)

        def bbody(eb, _):
            pltpu.sync_copy(ed_hbm.at[pl.ds(eb * 4 * EPW, 4 * EPW)], sbuf)

            def fbody(g, cnt):
                sv = sbuf[pl.ds(g * 16, 16)]
                dv = sbuf[pl.ds(EPW + g * 16, 16)]
                m = (dv >= lo) & (dv < lo + PR)
                plsc.store_compressed(csrcf.at[pl.ds(cnt, 16)], sv, mask=m)
                plsc.store_compressed(cdstf.at[pl.ds(cnt, 16)], dv - lo, mask=m)
                plsc.store_compressed(ca0.at[pl.ds(cnt, 16)],
                                      sbuf[pl.ds(2 * EPW + g * 16, 16)], mask=m)
                plsc.store_compressed(ca1.at[pl.ds(cnt, 16)],
                                      sbuf[pl.ds(3 * EPW + g * 16, 16)], mask=m)
                return cnt + jnp.sum(m.astype(jnp.int32))
            cnt = lax.fori_loop(0, NG, fbody, 0)

            csrcf[pl.ds(cnt, 16)] = iz16
            cdstf[pl.ds(cnt, 16)] = iz16
            ca0[pl.ds(cnt, 16)] = iz16
            ca1[pl.ds(cnt, 16)] = iz16
            ng = (cnt + 15) // 16

            def mbody(g, _):
                pltpu.async_copy(
                    xp_hbm.at[csrcf.at[pl.ds(g * 16, 16)]], rows, sem).wait()
                dlocv = cdstf[pl.ds(g * 16, 16)]
                av0 = plsc.bitcast(ca0[pl.ds(g * 16, 16)], jnp.float32)
                av1 = plsc.bitcast(ca1[pl.ds(g * 16, 16)], jnp.float32)

                def ebody(l, _):
                    db = _bcast_lane_i32(dlocv, l)
                    b0 = _bcast_lane(av0, l)
                    b1 = _bcast_lane(av1, l)
                    for cc in range(FP // 16):
                        v = rows[l, pl.ds(cc * 16, 16)] * (b0 if cc < FP // 32 else b1)
                        plsc.addupdate_scatter(acc, [db, cc * 16 + lanes], v)
                    return 0
                lax.fori_loop(0, 16, ebody, 0)
                return 0
            lax.fori_loop(0, ng, mbody, 0)
            return 0
        lax.fori_loop(0, 32, bbody, 0)

        pltpu.sync_copy(acc, part_hbm.at[pl.ds(lo, PR), :])


def _gat_sc(x, src_p, dst_p, W_arr, A_vec):
    bn = 1000
    xp, atab = pl.pallas_call(
        _xp_body,
        grid=(N // bn,),
        in_specs=[
            pl.BlockSpec((bn, x.shape[1]), lambda i: (i, 0)),
            pl.BlockSpec((x.shape[1], FP), lambda i: (0, 0)),
            pl.BlockSpec((FP, 4), lambda i: (0, 0)),
        ],
        out_specs=[
            pl.BlockSpec((bn, FP), lambda i: (i, 0)),
            pl.BlockSpec((bn, 4), lambda i: (i, 0)),
        ],
        out_shape=[
            jax.ShapeDtypeStruct((N, FP), jnp.float32),
            jax.ShapeDtypeStruct((N, 4), jnp.float32),
        ],
    )(x, W_arr, A_vec)

    k2 = functools.partial(
        pl.kernel,
        out_type=[
            jax.ShapeDtypeStruct((4 * EP,), jnp.int32),
            jax.ShapeDtypeStruct((32 * DTOT,), jnp.float32),
        ],
        mesh=plsc.VectorSubcoreMesh(core_axis_name="c", subcore_axis_name="s"),
        compiler_params=pltpu.CompilerParams(needs_layout_passes=False),
        scratch_types=[
            pltpu.VMEM((EPW,), jnp.int32),
            pltpu.VMEM((EPW,), jnp.int32),
            pltpu.VMEM((EPW,), jnp.int32),
            pltpu.VMEM((EPW,), jnp.int32),
            pltpu.VMEM((ATOT,), jnp.float32),
            pltpu.VMEM((DTOT,), jnp.float32),
            pltpu.SemaphoreType.DMA,
        ],
    )(_edge_ee_body)
    edata, dn_part = k2(src_p, dst_p, jnp.pad(atab.reshape(-1), (0, ATOT - 4 * N)))

    k3 = functools.partial(
        pl.kernel,
        out_type=jax.ShapeDtypeStruct((DTOT,), jnp.float32),
        mesh=plsc.VectorSubcoreMesh(core_axis_name="c", subcore_axis_name="s"),
        compiler_params=pltpu.CompilerParams(needs_layout_passes=False),
        scratch_types=[
            pltpu.VMEM((DTOT,), jnp.float32),
            pltpu.VMEM((DTOT // 32,), jnp.float32),
            pltpu.SemaphoreType.DMA,
        ],
    )(_denom_reduce_body)
    denom = k3(dn_part)

    k4 = functools.partial(
        pl.kernel,
        out_type=jax.ShapeDtypeStruct((NP, FP), jnp.float32),
        mesh=plsc.VectorSubcoreMesh(core_axis_name="c", subcore_axis_name="s"),
        compiler_params=pltpu.CompilerParams(needs_layout_passes=False),
        scratch_types=[
            pltpu.VMEM((4 * EPW,), jnp.int32),
            pltpu.VMEM((CBUF,), jnp.int32),
            pltpu.VMEM((CBUF,), jnp.int32),
            pltpu.VMEM((CBUF,), jnp.int32),
            pltpu.VMEM((CBUF,), jnp.int32),
            pltpu.VMEM((PR, FP), jnp.float32),
            pltpu.VMEM((16, FP), jnp.float32),
            pltpu.SemaphoreType.DMA,
        ],
    )(_msg_body)
    part = k4(edata, xp)
    return part, denom


def _attention(hpad, q, W_lin_pad, b_lin_pad):
    s, m, z = pl.pallas_call(
        functools.partial(_stats_body, ni=NI, bi=BI),
        grid=(NJ, NI),
        in_specs=[
            pl.BlockSpec((BI, FP), lambda j, i: (i, 0)),
            pl.BlockSpec((BJ, FP), lambda j, i: (j, 0)),
        ],
        out_specs=[
            pl.BlockSpec((BI, BJ), lambda j, i: (i, j)),
            pl.BlockSpec((1, BJ), lambda j, i: (0, j)),
            pl.BlockSpec((1, BJ), lambda j, i: (0, j)),
        ],
        out_shape=[
            jax.ShapeDtypeStruct((NP, NP), jnp.float32),
            jax.ShapeDtypeStruct((1, NP), jnp.float32),
            jax.ShapeDtypeStruct((1, NP), jnp.float32),
        ],
        scratch_shapes=[
            pltpu.VMEM((1, BJ), jnp.float32),
            pltpu.VMEM((1, BJ), jnp.float32),
        ],
        compiler_params=pltpu.CompilerParams(
            dimension_semantics=("parallel", "arbitrary")),
    )(q, hpad)

    vprime = hpad / z.reshape(NP, 1)

    out_pad = pl.pallas_call(
        functools.partial(_attn_body, nj=NJ),
        grid=(NI, NJ),
        in_specs=[
            pl.BlockSpec((BI, BJ), lambda i, j: (i, j)),
            pl.BlockSpec((1, BJ), lambda i, j: (0, j)),
            pl.BlockSpec((BJ, FP), lambda i, j: (j, 0)),
            pl.BlockSpec((FP, CP), lambda i, j: (0, 0)),
            pl.BlockSpec((1, CP), lambda i, j: (0, 0)),
        ],
        out_specs=pl.BlockSpec((BI, CP), lambda i, j: (i, 0)),
        out_shape=jax.ShapeDtypeStruct((NP, CP), jnp.float32),
        scratch_shapes=[pltpu.VMEM((BI, FP), jnp.float32)],
        compiler_params=pltpu.CompilerParams(
            dimension_semantics=("parallel", "arbitrary")),
    )(s, m, vprime, W_lin_pad, b_lin_pad.reshape(1, CP))
    return out_pad


def kernel(features, edges, W_gat, a_src, a_dst, b_gat, W_att, W_lin, b_lin):
    src = edges[0]
    dst = edges[1]

    src_p = jnp.concatenate([src, jnp.zeros((EP - E,), jnp.int32)])
    dst_p = jnp.concatenate([dst, jnp.full((EP - E,), N, jnp.int32)])

    W_arr = jnp.pad(W_gat, ((0, 0), (0, 0), (0, 20))).transpose(1, 0, 2).reshape(300, FP)
    a_src_p = jnp.pad(a_src, ((0, 0), (0, 20)))
    a_dst_p = jnp.pad(a_dst, ((0, 0), (0, 20)))
    z320 = jnp.zeros((320,), jnp.float32)
    A_vec = jnp.stack([
        jnp.concatenate([a_src_p[0], z320]),
        jnp.concatenate([z320, a_src_p[1]]),
        jnp.concatenate([a_dst_p[0], z320]),
        jnp.concatenate([z320, a_dst_p[1]]),
    ], axis=1)

    b_gat_pad = jnp.pad(b_gat.reshape(2, 300), ((0, 0), (0, 20))).reshape(1, FP)
    W_att_pad = jnp.pad(W_att.reshape(2, 300, 2, 300),
                        ((0, 0), (0, 20), (0, 0), (0, 20))).reshape(FP, FP)
    W_lin_pad = jnp.pad(W_lin.reshape(2, 300, CLASSES),
                        ((0, 0), (0, 20), (0, CP - CLASSES))).reshape(FP, CP)
    b_lin_pad = jnp.concatenate([b_lin, jnp.full((CP - CLASSES,), -1e30, jnp.float32)])

    part, denom = _gat_sc(features, src_p, dst_p, W_arr, A_vec)
    denom_t = denom.reshape(2, NP).T

    bn = 1024
    hpad, q = pl.pallas_call(
        functools.partial(_hq_body, bn=bn),
        grid=(NP // bn,),
        in_specs=[
            pl.BlockSpec((bn, FP), lambda i: (i, 0)),
            pl.BlockSpec((bn, 2), lambda i: (i, 0)),
            pl.BlockSpec((1, FP), lambda i: (0, 0)),
            pl.BlockSpec((FP, FP), lambda i: (0, 0)),
        ],
        out_specs=[
            pl.BlockSpec((bn, FP), lambda i: (i, 0)),
            pl.BlockSpec((bn, FP), lambda i: (i, 0)),
        ],
        out_shape=[
            jax.ShapeDtypeStruct((NP, FP), jnp.float32),
            jax.ShapeDtypeStruct((NP, FP), jnp.float32),
        ],
    )(part, denom_t, b_gat_pad, W_att_pad)

    out_pad = _attention(hpad, q, W_lin_pad, b_lin_pad)
    return out_pad[:N, :CLASSES]
